# baseline (speedup 1.0000x reference)
import functools

import jax
import jax.numpy as jnp
from jax import lax
from jax.experimental import pallas as pl
from jax.experimental.pallas import tpu as pltpu
from jax.experimental.pallas import tpu_sc as plsc

_BINS = (-1.8, -1.5429, -1.2857, -1.0286, -0.7714, -0.5143, -0.2571, 0.0,
         0.2571, 0.5143, 0.7714, 1.0286, 1.2857, 1.5429, 1.8)
_NG = 16
_N = 4096
_L = 16
_H = _N // 2
_HCH = _H // _L
_MSL = _N // 32
_NEG = float("-inf")
_POS = float("inf")


@jax.jit
def _sc_dispatch(y):
    mesh = plsc.VectorSubcoreMesh(
        core_axis_name="c", subcore_axis_name="s", num_cores=2, num_subcores=16
    )

    @functools.partial(
        pl.kernel,
        out_type=(
            jax.ShapeDtypeStruct((_N,), jnp.int32),
            jax.ShapeDtypeStruct((_N,), jnp.int32),
            jax.ShapeDtypeStruct((_NG,), jnp.int32),
        ),
        mesh=mesh,
        compiler_params=pltpu.CompilerParams(needs_layout_passes=False),
        scratch_types=[
            pltpu.VMEM((_N,), jnp.float32),
            pltpu.VMEM((_MSL,), jnp.int32),
            pltpu.VMEM((_H + _L,), jnp.int32),
            pltpu.VMEM((_H,), jnp.int32),
            pltpu.VMEM((_H,), jnp.int32),
            pltpu.VMEM((_L,), jnp.int32),
            pltpu.SemaphoreType.DMA,
        ],
    )
    def k(y_hbm, m_hbm, order_hbm, counts_hbm,
          y_loc, m_sl, comp, vals, tgt, tmp_row, sem):
        g = lax.axis_index("s")
        h = lax.axis_index("c")
        pltpu.sync_copy(y_hbm, y_loc)

        iota = lax.iota(jnp.int32, _L)
        my_base = pl.multiple_of(h * _H, _H)
        other_base = pl.multiple_of((1 - h) * _H, _H)

        blo = jnp.float32(_NEG)
        for kk in range(1, _NG):
            blo = jnp.where(g == kk, jnp.float32(_BINS[kk - 1]), blo)
        bhi = jnp.float32(_POS)
        for kk in range(_NG - 1):
            bhi = jnp.where(g == kk, jnp.float32(_BINS[kk]), bhi)

        def other_body(kk, carry):
            ltv, eqov = carry
            yv = y_loc[pl.ds(other_base + kk * _L, _L)]
            ltv = ltv + jnp.where(yv <= blo, 1, 0)
            eqov = eqov + jnp.where((yv > blo) & (yv <= bhi), 1, 0)
            return ltv, eqov

        z = jnp.zeros((_L,), jnp.int32)
        ltv, eqov = lax.fori_loop(0, _HCH, other_body, (z, z))

        def my_body(kk, carry):
            ltv, cbase = carry
            yv = y_loc[pl.ds(my_base + kk * _L, _L)]
            eqm = (yv > blo) & (yv <= bhi)
            eqi = jnp.where(eqm, 1, 0)
            pc = plsc.cumsum(eqi)
            pos = jnp.where(eqm, cbase + pc - 1, _H + iota)
            plsc.store_scatter(comp, [pos], my_base + kk * _L + iota)
            cbase = cbase + plsc.all_reduce_population_count(eqm)
            ltv = ltv + jnp.where(yv <= blo, 1, 0)
            return ltv, cbase

        ltv, cbase = lax.fori_loop(0, _HCH, my_body, (ltv, z))

        c = jnp.max(cbase)
        eqo_n = jnp.sum(eqov)
        off = jnp.sum(ltv) + jnp.where(h == 1, eqo_n, 0)

        msl_base = pl.multiple_of(my_base + g * _MSL, _MSL)

        def m_body(kk, _):
            yv = y_loc[pl.ds(msl_base + kk * _L, _L)]
            m = jnp.where(yv > _BINS[0], 1, 0)
            for b in _BINS[1:]:
                m = m + jnp.where(yv > b, 1, 0)
            m_sl[pl.ds(kk * _L, _L)] = m
            return 0

        lax.fori_loop(0, _MSL // _L, m_body, 0)
        pltpu.sync_copy(m_sl, m_hbm.at[pl.ds(msl_base, _MSL)])

        @pl.when(h == 1)
        def _():
            tmp_row[...] = z + (eqo_n + c)
            pltpu.async_copy(tmp_row, counts_hbm.at[z + g], sem).wait()

        @pl.when(c > 0)
        def _():
            def tgt_body(kk, _):
                jc = jnp.minimum(kk * _L + iota, c - 1)
                vals[pl.ds(kk * _L, _L)] = plsc.load_gather(comp, [jc])
                tgt[pl.ds(kk * _L, _L)] = off + jc
                return 0

            lax.fori_loop(0, _HCH, tgt_body, 0)
            pltpu.sync_copy(vals, order_hbm.at[tgt])

    return k(y)


def kernel(x):
    y = x[0, :, 0]
    memberships, order, counts = _sc_dispatch(y)
    return memberships, order, counts


# speedup vs baseline: 26.9620x; 26.9620x over previous
import functools

import jax
import jax.numpy as jnp
from jax import lax
from jax.experimental import pallas as pl
from jax.experimental.pallas import tpu as pltpu
from jax.experimental.pallas import tpu_sc as plsc

_BINS = (-1.8, -1.5429, -1.2857, -1.0286, -0.7714, -0.5143, -0.2571, 0.0,
         0.2571, 0.5143, 0.7714, 1.0286, 1.2857, 1.5429, 1.8)
_NG = 16
_N = 4096
_L = 16
_NCHUNK = _N // _L
_MSL = _N // _NG
_UNROLL = 2


@jax.jit
def _sc_dispatch(y):
    mesh = plsc.VectorSubcoreMesh(
        core_axis_name="c", subcore_axis_name="s", num_cores=2, num_subcores=16
    )

    @functools.partial(
        pl.kernel,
        out_type=(
            jax.ShapeDtypeStruct((_N,), jnp.int32),
            jax.ShapeDtypeStruct((_N,), jnp.int32),
            jax.ShapeDtypeStruct((_NG,), jnp.int32),
        ),
        mesh=mesh,
        compiler_params=pltpu.CompilerParams(needs_layout_passes=False),
        scratch_types=[
            pltpu.VMEM((_N,), jnp.float32),
            pltpu.VMEM((_MSL,), jnp.int32),
            pltpu.VMEM((_N + _L,), jnp.int32),
            pltpu.VMEM((_N,), jnp.int32),
            pltpu.VMEM((_L,), jnp.int32),
            pltpu.VMEM((_NG * _L,), jnp.int32),
            pltpu.VMEM((_MSL,), jnp.int32),
            pltpu.VMEM_SHARED((_N + _NG,), jnp.int32),
            pltpu.VMEM_SHARED((_NG * _L,), jnp.int32),
        ],
    )
    def k(y_hbm, m_hbm, order_hbm, counts_hbm,
          y_loc, m_sl, comp, tgt, tmp_row, cnt_loc, bounce, order_sh, cnt_sh):
        g = lax.axis_index("s")
        h = lax.axis_index("c")
        iota = lax.iota(jnp.int32, _L)
        base = pl.multiple_of(g * _MSL, _MSL)

        @pl.when(h == 1)
        def _():
            pltpu.sync_copy(y_hbm.at[pl.ds(base, _MSL)], y_loc.at[pl.ds(0, _MSL)])

            def m_body(kk, _):
                yv = y_loc[pl.ds(kk * _L, _L)]
                m = jnp.where(yv > _BINS[0], 1, 0)
                for b in _BINS[1:]:
                    m = m + jnp.where(yv > b, 1, 0)
                m_sl[pl.ds(kk * _L, _L)] = m
                return 0

            lax.fori_loop(0, _MSL // _L, m_body, 0)
            pltpu.sync_copy(m_sl, m_hbm.at[pl.ds(base, _MSL)])

        @pl.when(h == 0)
        def _():
            pltpu.sync_copy(y_hbm, y_loc)

            blo = jnp.float32(float("-inf"))
            for kk in range(1, _NG):
                blo = jnp.where(g == kk, jnp.float32(_BINS[kk - 1]), blo)
            bhi = jnp.float32(float("inf"))
            for kk in range(_NG - 1):
                bhi = jnp.where(g == kk, jnp.float32(_BINS[kk]), bhi)

            z = jnp.zeros((_L,), jnp.int32)

            def scan_body(kk, carry):
                ltv, cbase = carry
                for u in range(_UNROLL):
                    j0 = (kk * _UNROLL + u) * _L
                    yv = y_loc[pl.ds(j0, _L)]
                    eqm = (yv > blo) & (yv <= bhi)
                    pc = plsc.cumsum(jnp.where(eqm, 1, 0))
                    pos = jnp.where(eqm, cbase + pc - 1, _N + iota)
                    plsc.store_scatter(comp, [pos], j0 + iota)
                    cbase = cbase + plsc.all_reduce_population_count(eqm)
                    ltv = ltv + jnp.where(yv <= blo, 1, 0)
                return ltv, cbase

            ltv, cbase = lax.fori_loop(
                0, _NCHUNK // _UNROLL, scan_body, (z, z))
            c = jnp.max(cbase)
            off = jnp.sum(ltv)

            def tgt_body(kk, _):
                for u in range(_UNROLL):
                    j0 = (kk * _UNROLL + u) * _L
                    j = j0 + iota
                    tgt[pl.ds(j0, _L)] = jnp.where(j < c, off + j, _N + g)
                return 0

            lax.fori_loop(0, _NCHUNK // _UNROLL, tgt_body, 0)
            pltpu.sync_copy(comp.at[pl.ds(0, _N)], order_sh.at[tgt])

            tmp_row[...] = z + c
            gb = pl.multiple_of(g * _L, _L)
            pltpu.sync_copy(tmp_row, cnt_sh.at[pl.ds(gb, _L)])

            plsc.subcore_barrier()

            pltpu.sync_copy(order_sh.at[pl.ds(base, _MSL)], bounce)
            pltpu.sync_copy(bounce, order_hbm.at[pl.ds(base, _MSL)])

            @pl.when(g == 0)
            def _():
                pltpu.sync_copy(cnt_sh, cnt_loc)
                counts = jnp.zeros((_L,), jnp.int32)
                for l in range(_NG):
                    counts = jnp.where(
                        iota == l, cnt_loc[pl.ds(l * _L, _L)], counts)
                tmp_row[...] = counts
                pltpu.sync_copy(tmp_row, counts_hbm)

    return k(y)


def kernel(x):
    y = x[0, :, 0]
    memberships, order, counts = _sc_dispatch(y)
    return memberships, order, counts
